# baseline (speedup 1.0000x reference)
import functools

import jax
import jax.numpy as jnp
from jax import lax
from jax.experimental import pallas as pl
from jax.experimental.pallas import tpu as pltpu
from jax.experimental.pallas import tpu_sc as plsc

C = 768
HW = 1024
N = 18

NUM_CORES = 2
NUM_SUBCORES = 16
NW = NUM_CORES * NUM_SUBCORES
CPW = C // NW
LANES = 16
VECS_PER_CH = HW // LANES


def _reduce_channel(buf, ch):
    z = jnp.zeros((LANES,), jnp.float32)

    def body(j, accs):
        a0, a1, a2, a3 = accs
        base = j * (4 * LANES)
        a0 = a0 + buf[ch, pl.ds(base, LANES)]
        a1 = a1 + buf[ch, pl.ds(base + LANES, LANES)]
        a2 = a2 + buf[ch, pl.ds(base + 2 * LANES, LANES)]
        a3 = a3 + buf[ch, pl.ds(base + 3 * LANES, LANES)]
        return (a0, a1, a2, a3)

    a0, a1, a2, a3 = lax.fori_loop(0, VECS_PER_CH // 4, body, (z, z, z, z),
                                   unroll=4)
    return jnp.sum((a0 + a1) + (a2 + a3))


def _pool_slab(buf, mv, mv_off):
    inv = jnp.float32(1.0 / HW)
    lane = lax.iota(jnp.int32, LANES)
    for g in range(0, CPW, LANES):
        vec = jnp.zeros((LANES,), jnp.float32)
        for ch in range(g, min(g + LANES, CPW)):
            s = _reduce_channel(buf, ch) * inv
            vec = jnp.where(lane == (ch - g), s, vec)
        mv[pl.ds(mv_off + g, LANES)] = vec


@functools.partial(
    pl.kernel,
    out_type=jax.ShapeDtypeStruct((2 * C,), jnp.float32),
    mesh=plsc.VectorSubcoreMesh(core_axis_name="c", subcore_axis_name="s"),
    scratch_types=[
        pltpu.VMEM((CPW, HW), jnp.float32),
        pltpu.VMEM((CPW, HW), jnp.float32),
        pltpu.VMEM((4 * LANES,), jnp.float32),
    ],
    compiler_params=pltpu.CompilerParams(needs_layout_passes=False),
)
def _sc_pool(x_hbm, c_hbm, out_hbm, xv, cv, mv):
    w = lax.axis_index("s") * NUM_CORES + lax.axis_index("c")
    c0 = w * CPW
    pltpu.sync_copy(x_hbm.at[pl.ds(c0, CPW), :], xv)
    pltpu.sync_copy(c_hbm.at[pl.ds(c0, CPW), :], cv)
    _pool_slab(xv, mv, 0)
    _pool_slab(cv, mv, 2 * LANES)
    pltpu.sync_copy(mv.at[pl.ds(0, CPW)], out_hbm.at[pl.ds(c0, CPW)])
    pltpu.sync_copy(mv.at[pl.ds(2 * LANES, CPW)], out_hbm.at[pl.ds(C + c0, CPW)])


def _tc_graph_body(nidx_ref, pooled_ref, mem_ref, img_ref, adj_ref):
    m = mem_ref[...]
    fx = pooled_ref[pl.ds(0, 1), :]
    fc = pooled_ref[pl.ds(1, 1), :]
    rows = lax.broadcasted_iota(jnp.int32, (N, 1), 0)
    m = jnp.where(rows == nidx_ref[0], fx, m)
    m = jnp.where(rows == nidx_ref[1], fc, m)
    ss = jnp.sum(m * m, axis=1, keepdims=True)
    nrm = m / (jnp.sqrt(ss) + 1e-8)
    adj = lax.dot_general(nrm, nrm, (((1,), (1,)), ((), ())),
                          preferred_element_type=jnp.float32)
    adj_ref[...] = adj[None]
    img_ref[...] = jnp.sum(m, axis=0, keepdims=True) * jnp.float32(1.0 / N)


_tc_graph = pl.pallas_call(
    _tc_graph_body,
    out_shape=(
        jax.ShapeDtypeStruct((1, C), jnp.float32),
        jax.ShapeDtypeStruct((1, N, N), jnp.float32),
    ),
    in_specs=[
        pl.BlockSpec(memory_space=pltpu.SMEM),
        pl.BlockSpec(memory_space=pltpu.VMEM),
        pl.BlockSpec(memory_space=pltpu.VMEM),
    ],
    out_specs=(
        pl.BlockSpec(memory_space=pltpu.VMEM),
        pl.BlockSpec(memory_space=pltpu.VMEM),
    ),
)


def kernel(x, conds, mem, node_idx):
    x2 = x.reshape(C, HW)
    c2 = conds.reshape(C, HW)
    pooled = _sc_pool(x2, c2).reshape(2, C)
    img_feature, adjacency = _tc_graph(node_idx, pooled, mem)
    return (img_feature, adjacency)


# speedup vs baseline: 1.1035x; 1.1035x over previous
import functools

import jax
import jax.numpy as jnp
from jax import lax
from jax.experimental import pallas as pl
from jax.experimental.pallas import tpu as pltpu
from jax.experimental.pallas import tpu_sc as plsc

C = 768
HW = 1024
N = 18

NUM_CORES = 2
NUM_SUBCORES = 16
NW = NUM_CORES * NUM_SUBCORES
CPW = C // NW
LANES = 16
VECS_PER_CH = HW // LANES


N_ACC = 8


def _reduce_channel(buf, ch):
    accs = [jnp.zeros((LANES,), jnp.float32) for _ in range(N_ACC)]
    for j in range(VECS_PER_CH):
        accs[j % N_ACC] = accs[j % N_ACC] + buf[ch, pl.ds(j * LANES, LANES)]
    while len(accs) > 1:
        accs = [accs[i] + accs[i + 1] for i in range(0, len(accs), 2)]
    v = accs[0]
    lane = lax.iota(jnp.int32, LANES)
    for sh in (8, 4, 2, 1):
        v = v + v.at[lane ^ sh].get(mode="promise_in_bounds")
    return v


def _pool_slab(buf, mv, mv_off):
    inv = jnp.float32(1.0 / HW)
    lane = lax.iota(jnp.int32, LANES)
    mask0 = lane == 0
    for ch in range(CPW):
        s = _reduce_channel(buf, ch) * inv
        idx = jnp.full((LANES,), mv_off + ch, jnp.int32)
        plsc.store_scatter(mv, [idx], s, mask=mask0)


@functools.partial(
    pl.kernel,
    out_type=jax.ShapeDtypeStruct((2 * C,), jnp.float32),
    mesh=plsc.VectorSubcoreMesh(core_axis_name="c", subcore_axis_name="s"),
    scratch_types=[
        pltpu.VMEM((CPW, HW), jnp.float32),
        pltpu.VMEM((CPW, HW), jnp.float32),
        pltpu.VMEM((4 * LANES,), jnp.float32),
        pltpu.SemaphoreType.DMA,
        pltpu.SemaphoreType.DMA,
    ],
    compiler_params=pltpu.CompilerParams(needs_layout_passes=False),
)
def _sc_pool(x_hbm, c_hbm, out_hbm, xv, cv, mv, sem_x, sem_c):
    w = lax.axis_index("s") * NUM_CORES + lax.axis_index("c")
    c0 = w * CPW
    hx = pltpu.async_copy(x_hbm.at[pl.ds(c0, CPW), :], xv, sem_x)
    hc = pltpu.async_copy(c_hbm.at[pl.ds(c0, CPW), :], cv, sem_c)
    hx.wait()
    _pool_slab(xv, mv, 0)
    hc.wait()
    _pool_slab(cv, mv, 2 * LANES)
    pltpu.sync_copy(mv.at[pl.ds(0, CPW)], out_hbm.at[pl.ds(c0, CPW)])
    pltpu.sync_copy(mv.at[pl.ds(2 * LANES, CPW)], out_hbm.at[pl.ds(C + c0, CPW)])


def _tc_graph_body(nidx_ref, pooled_ref, mem_ref, img_ref, adj_ref):
    m = mem_ref[...]
    fx = pooled_ref[pl.ds(0, 1), :]
    fc = pooled_ref[pl.ds(1, 1), :]
    rows = lax.broadcasted_iota(jnp.int32, (N, 1), 0)
    m = jnp.where(rows == nidx_ref[0], fx, m)
    m = jnp.where(rows == nidx_ref[1], fc, m)
    ss = jnp.sum(m * m, axis=1, keepdims=True)
    nrm = m / (jnp.sqrt(ss) + 1e-8)
    adj = lax.dot_general(nrm, nrm, (((1,), (1,)), ((), ())),
                          preferred_element_type=jnp.float32)
    adj_ref[...] = adj[None]
    img_ref[...] = jnp.sum(m, axis=0, keepdims=True) * jnp.float32(1.0 / N)


_tc_graph = pl.pallas_call(
    _tc_graph_body,
    out_shape=(
        jax.ShapeDtypeStruct((1, C), jnp.float32),
        jax.ShapeDtypeStruct((1, N, N), jnp.float32),
    ),
    in_specs=[
        pl.BlockSpec(memory_space=pltpu.SMEM),
        pl.BlockSpec(memory_space=pltpu.VMEM),
        pl.BlockSpec(memory_space=pltpu.VMEM),
    ],
    out_specs=(
        pl.BlockSpec(memory_space=pltpu.VMEM),
        pl.BlockSpec(memory_space=pltpu.VMEM),
    ),
)


def kernel(x, conds, mem, node_idx):
    x2 = x.reshape(C, HW)
    c2 = conds.reshape(C, HW)
    pooled = _sc_pool(x2, c2).reshape(2, C)
    img_feature, adjacency = _tc_graph(node_idx, pooled, mem)
    return (img_feature, adjacency)


# speedup vs baseline: 1.4952x; 1.3550x over previous
import functools

import jax
import jax.numpy as jnp
from jax import lax
from jax.experimental import pallas as pl
from jax.experimental.pallas import tpu as pltpu
from jax.experimental.pallas import tpu_sc as plsc

C = 768
H = 32
W = 32
HW = H * W
N = 18

NUM_CORES = 2
NUM_SUBCORES = 16
NW = NUM_CORES * NUM_SUBCORES
PPW = HW // NW
LANES = 16
CHUNKS = C // LANES


def _accum_slab(buf, pv):
    def body(k, carry):
        off = k * LANES
        acc = buf[0, pl.ds(off, LANES)]
        for p in range(1, PPW):
            acc = acc + buf[p, pl.ds(off, LANES)]
        pv[pl.ds(off, LANES)] = acc
        return carry

    lax.fori_loop(0, CHUNKS, body, 0, unroll=2)


@functools.partial(
    pl.kernel,
    out_type=jax.ShapeDtypeStruct((2 * NW * C,), jnp.float32),
    mesh=plsc.VectorSubcoreMesh(core_axis_name="c", subcore_axis_name="s"),
    scratch_types=[
        pltpu.VMEM((PPW, C), jnp.float32),
        pltpu.VMEM((PPW, C), jnp.float32),
        pltpu.VMEM((C,), jnp.float32),
        pltpu.SemaphoreType.DMA,
        pltpu.SemaphoreType.DMA,
    ],
)
def _sc_pool(x_hbm, c_hbm, out_hbm, xv, cv, pv, sem_x, sem_c):
    w = lax.axis_index("s") * NUM_CORES + lax.axis_index("c")
    p0 = w * PPW
    hx = pltpu.async_copy(x_hbm.at[pl.ds(p0, PPW), :], xv, sem_x)
    hc = pltpu.async_copy(c_hbm.at[pl.ds(p0, PPW), :], cv, sem_c)
    hx.wait()
    _accum_slab(xv, pv)
    pltpu.sync_copy(pv, out_hbm.at[pl.ds(w * C, C)])
    hc.wait()
    _accum_slab(cv, pv)
    pltpu.sync_copy(pv, out_hbm.at[pl.ds((NW + w) * C, C)])


def _sum_parts(parts_ref, base):
    acc = parts_ref[pl.ds(base * C, C)]
    for i in range(1, NW):
        acc = acc + parts_ref[pl.ds((base + i) * C, C)]
    return acc


def _tc_graph_body(nidx_ref, parts_ref, mem_ref, img_ref, adj_ref):
    inv = jnp.float32(1.0 / HW)
    fx = jnp.reshape(_sum_parts(parts_ref, 0) * inv, (1, C))
    fc = jnp.reshape(_sum_parts(parts_ref, NW) * inv, (1, C))
    m = mem_ref[...]
    rows = lax.broadcasted_iota(jnp.int32, (N, 1), 0)
    m = jnp.where(rows == nidx_ref[0], fx, m)
    m = jnp.where(rows == nidx_ref[1], fc, m)
    ss = jnp.sum(m * m, axis=1, keepdims=True)
    nrm = m / (jnp.sqrt(ss) + 1e-8)
    adj = lax.dot_general(nrm, nrm, (((1,), (1,)), ((), ())),
                          preferred_element_type=jnp.float32)
    adj_ref[...] = adj[None]
    img_ref[...] = jnp.sum(m, axis=0, keepdims=True) * jnp.float32(1.0 / N)


_tc_graph = pl.pallas_call(
    _tc_graph_body,
    out_shape=(
        jax.ShapeDtypeStruct((1, C), jnp.float32),
        jax.ShapeDtypeStruct((1, N, N), jnp.float32),
    ),
    in_specs=[
        pl.BlockSpec(memory_space=pltpu.SMEM),
        pl.BlockSpec(memory_space=pltpu.VMEM),
        pl.BlockSpec(memory_space=pltpu.VMEM),
    ],
    out_specs=(
        pl.BlockSpec(memory_space=pltpu.VMEM),
        pl.BlockSpec(memory_space=pltpu.VMEM),
    ),
)


def kernel(x, conds, mem, node_idx):
    xt = x.transpose(1, 2, 0).reshape(HW, C)
    ct = conds.transpose(1, 2, 0).reshape(HW, C)
    parts = _sc_pool(xt, ct)
    img_feature, adjacency = _tc_graph(node_idx, parts, mem)
    return (img_feature, adjacency)
